# baseline (speedup 1.0000x reference)
import functools
import math

import jax
import jax.numpy as jnp
from jax import lax
from jax.experimental import pallas as pl
from jax.experimental.pallas import tpu as pltpu
from jax.experimental.pallas import tpu_sc as plsc

_N = 5000
_TMAX = 50
_CHUNK = 256
_NPAD = 5120
_PADVAL = 1.0e9

_NSUB = 16
_EV_PER_SUB = _NPAD // _NSUB
_NBINS = 64


def _sc_splat(vec, lane):
    dnums = lax.GatherDimensionNumbers(
        offset_dims=(), collapsed_slice_dims=(0,), start_index_map=(0,))
    idx = jnp.full((16, 1), lane, dtype=jnp.int32)
    return lax.gather(vec, idx, dnums, (1,),
                      mode=lax.GatherScatterMode.PROMISE_IN_BOUNDS)


def _sc_segment_body(t_hbm, par_hbm, out_hbm, tloc, hloc, parv, allh, outv,
                     shared):
    cid = lax.axis_index("c")
    sid = lax.axis_index("s")

    base = sid * _EV_PER_SUB
    pltpu.sync_copy(t_hbm.at[pl.ds(base, _EV_PER_SUB)], tloc)
    pltpu.sync_copy(par_hbm, parv)

    iota = lax.iota(jnp.int32, 16)
    binv = [(iota + (1 + 16 * c)).astype(jnp.float32) for c in range(4)]

    def ev_body(i, hs):
        ev = tloc[pl.ds(i * 16, 16)]
        out = list(hs)
        for j in range(16):
            e = _sc_splat(ev, j)
            for c in range(4):
                out[c] = out[c] + jnp.where(binv[c] == e, 1.0, 0.0)
        return tuple(out)

    zeros16 = jnp.zeros((16,), jnp.float32)
    h = lax.fori_loop(0, _EV_PER_SUB // 16, ev_body,
                      (zeros16, zeros16, zeros16, zeros16))
    for c in range(4):
        hloc[pl.ds(16 * c, 16)] = h[c]
    pltpu.sync_copy(hloc, shared.at[pl.ds(sid * _NBINS, _NBINS)])
    plsc.subcore_barrier()

    @pl.when((sid == 0) & (cid == 0))
    def _():
        pltpu.sync_copy(shared, allh)
        g = [zeros16, zeros16, zeros16, zeros16]
        for w in range(_NSUB):
            for c in range(4):
                g[c] = g[c] + allh[pl.ds(w * _NBINS + 16 * c, 16)]

        pv = parv[pl.ds(0, 16)]
        c_v = _sc_splat(pv, 1)
        beta_v = _sc_splat(pv, 2)

        cand = [jnp.where(g[c] > 0.0, binv[c], 0.0) for c in range(4)]
        m = jnp.maximum(jnp.maximum(cand[0], cand[1]),
                        jnp.maximum(cand[2], cand[3]))
        nf_v = _sc_splat(m, 0)
        for k in range(1, 16):
            nf_v = jnp.maximum(nf_v, _sc_splat(m, k))
        c1_v = _sc_splat(g[0], 0)

        nz = [jnp.where(g[c] > 0.0, 1.0, 0.0) for c in range(4)]
        scum = []
        sprev = []
        carry = zeros16
        for c in range(4):
            cs = carry
            for k in range(16):
                cs = cs + jnp.where(iota >= k, _sc_splat(nz[c], k), 0.0)
            scum.append(cs)
            sprev.append(cs - nz[c])
            carry = _sc_splat(cs, 15)

        acc = zeros16
        for t in range(2, _TMAX + 1):
            gidx = t - 2
            st_v = _sc_splat(scum[gidx // 16], gidx % 16)
            wsum = zeros16
            for c in range(4):
                hi = t - 2 - 16 * c
                if hi < 0:
                    continue
                term = g[c] * jnp.exp(beta_v * (sprev[c] - st_v))
                if hi >= 15:
                    wsum = wsum + term
                else:
                    wsum = wsum + jnp.where(iota <= hi, term, 0.0)
            acc = acc + jnp.where(jnp.float32(t) <= nf_v, wsum, 0.0)
        tot_v = _sc_splat(acc, 0)
        for k in range(1, 16):
            tot_v = tot_v + _sc_splat(acc, k)
        tot_v = tot_v * c_v

        o = jnp.where(iota == 0, tot_v,
                      jnp.where(iota == 1, nf_v,
                                jnp.where(iota == 2, c1_v, zeros16)))
        outv[pl.ds(0, 16)] = o
        pltpu.sync_copy(outv, out_hbm)


_sc_segment_kernel = functools.partial(
    pl.kernel,
    out_type=jax.ShapeDtypeStruct((16,), jnp.float32),
    mesh=plsc.VectorSubcoreMesh(core_axis_name="c", subcore_axis_name="s"),
    compiler_params=pltpu.CompilerParams(needs_layout_passes=False),
    scratch_types=[
        pltpu.VMEM((_EV_PER_SUB,), jnp.float32),
        pltpu.VMEM((_NBINS,), jnp.float32),
        pltpu.VMEM((16,), jnp.float32),
        pltpu.VMEM((_NSUB * _NBINS,), jnp.float32),
        pltpu.VMEM((16,), jnp.float32),
        pltpu.VMEM_SHARED((_NSUB * _NBINS,), jnp.float32),
    ],
)(_sc_segment_body)


def _etas_kernel(tr, tc, xr, xc, yr, yc, par, o_logsum):
    lam0 = par[0, 0]
    c = par[0, 1]
    beta = par[0, 2]
    sx = par[0, 3]
    sy = par[0, 4]

    coef = c / (sx * sy * (2.0 * math.pi))
    inv_sx = 1.0 / (jnp.sqrt(jnp.float32(2.0)) * sx)
    inv_sy = 1.0 / (jnp.sqrt(jnp.float32(2.0)) * sy)
    nbeta = -beta

    tcv = tc[:, :]
    xcv = xc[:, :] * inv_sx
    ycv = yc[:, :] * inv_sy

    logsum = jnp.float32(0.0)
    for i in range(_NPAD // _CHUNK):
        ncol = (i + 1) * _CHUNK
        ta = tr[pl.ds(i * _CHUNK, _CHUNK), :]
        xa = xr[pl.ds(i * _CHUNK, _CHUNK), :] * inv_sx
        ya = yr[pl.ds(i * _CHUNK, _CHUNK), :] * inv_sy
        tb = tcv[:, :ncol]
        mask = tb < ta
        dt = jnp.where(mask, ta - tb, 1.0)
        r = 1.0 / dt
        dx = xa - xcv[:, :ncol]
        dy = ya - ycv[:, :ncol]
        s = dx * dx + dy * dy
        expo = nbeta * dt - s * r
        w = jnp.where(mask, jnp.exp(expo) * r, 0.0)
        lam = coef * jnp.sum(w, axis=1, keepdims=True)
        lmask = (ta >= 2.0) & (ta <= float(_TMAX))
        lam_safe = jnp.where(lmask, lam, 1.0)
        logsum = logsum + jnp.sum(jnp.where(lmask, jnp.log(lam_safe), 0.0))

    o_logsum[:, :] = jnp.reshape(logsum, (1, 1))


def kernel(obs, Lambda0, C, Beta, Sigmax, Sigmay):
    times = obs[:, 0]
    x = obs[:, 1]
    y = obs[:, 2]
    pad = _NPAD - _N
    tpad = jnp.pad(times, (0, pad), constant_values=_PADVAL)
    xpad = jnp.pad(x, (0, pad), constant_values=0.0)
    ypad = jnp.pad(y, (0, pad), constant_values=0.0)

    tr = tpad[:, None]
    tc = tpad[None, :]
    xr = xpad[:, None]
    xc = xpad[None, :]
    yr = ypad[:, None]
    yc = ypad[None, :]
    par = jnp.stack([Lambda0, C, Beta, Sigmax, Sigmay,
                     jnp.float32(0.0), jnp.float32(0.0), jnp.float32(0.0)])[None, :]
    par16 = jnp.concatenate(
        [par[0], jnp.zeros((8,), jnp.float32)], axis=0)

    sc_out = _sc_segment_kernel(tpad, par16)
    logsum = pl.pallas_call(
        _etas_kernel,
        out_shape=jax.ShapeDtypeStruct((1, 1), jnp.float32),
    )(tr, tc, xr, xc, yr, yc, par)[0, 0]

    total = sc_out[0]
    n_f = sc_out[1]
    count1 = sc_out[2]
    lams1 = count1 * jnp.log(Lambda0) + logsum
    lams2 = Lambda0 * n_f + total
    return (lams1 - lams2, lams1, lams2)


# speedup vs baseline: 1.1779x; 1.1779x over previous
import functools
import math

import jax
import jax.numpy as jnp
from jax import lax
from jax.experimental import pallas as pl
from jax.experimental.pallas import tpu as pltpu
from jax.experimental.pallas import tpu_sc as plsc

_N = 5000
_TMAX = 50
_CHUNK = 256
_NPAD = 5120
_PADVAL = 1.0e9

_NSUB = 16
_EV_PER_SUB = _NPAD // _NSUB
_NBINS = 64


def _sc_splat(vec, lane):
    dnums = lax.GatherDimensionNumbers(
        offset_dims=(), collapsed_slice_dims=(0,), start_index_map=(0,))
    idx = jnp.full((16, 1), lane, dtype=jnp.int32)
    return lax.gather(vec, idx, dnums, (1,),
                      mode=lax.GatherScatterMode.PROMISE_IN_BOUNDS)


def _sc_segment_body(t_hbm, par_hbm, out_hbm, tloc, hloc, parv, allh, outv,
                     shared):
    cid = lax.axis_index("c")
    sid = lax.axis_index("s")

    base = sid * _EV_PER_SUB
    pltpu.sync_copy(t_hbm.at[pl.ds(base, _EV_PER_SUB)], tloc)
    pltpu.sync_copy(par_hbm, parv)

    iota = lax.iota(jnp.int32, 16)
    binv = [(iota + (1 + 16 * c)).astype(jnp.float32) for c in range(4)]

    def ev_body(i, hs):
        ev = tloc[pl.ds(i * 16, 16)]
        out = list(hs)
        for j in range(16):
            e = _sc_splat(ev, j)
            for c in range(4):
                out[c] = out[c] + jnp.where(binv[c] == e, 1.0, 0.0)
        return tuple(out)

    zeros16 = jnp.zeros((16,), jnp.float32)
    h = lax.fori_loop(0, _EV_PER_SUB // 16, ev_body,
                      (zeros16, zeros16, zeros16, zeros16))
    for c in range(4):
        hloc[pl.ds(16 * c, 16)] = h[c]
    pltpu.sync_copy(hloc, shared.at[pl.ds(sid * _NBINS, _NBINS)])
    plsc.subcore_barrier()

    @pl.when((sid == 0) & (cid == 0))
    def _():
        pltpu.sync_copy(shared, allh)
        g = [zeros16, zeros16, zeros16, zeros16]
        for w in range(_NSUB):
            for c in range(4):
                g[c] = g[c] + allh[pl.ds(w * _NBINS + 16 * c, 16)]

        pv = parv[pl.ds(0, 16)]
        c_v = _sc_splat(pv, 1)
        beta_v = _sc_splat(pv, 2)

        cand = [jnp.where(g[c] > 0.0, binv[c], 0.0) for c in range(4)]
        m = jnp.maximum(jnp.maximum(cand[0], cand[1]),
                        jnp.maximum(cand[2], cand[3]))
        nf_v = _sc_splat(m, 0)
        for k in range(1, 16):
            nf_v = jnp.maximum(nf_v, _sc_splat(m, k))
        c1_v = _sc_splat(g[0], 0)

        nz = [jnp.where(g[c] > 0.0, 1.0, 0.0) for c in range(4)]
        scum = []
        sprev = []
        carry = zeros16
        for c in range(4):
            cs = carry
            for k in range(16):
                cs = cs + jnp.where(iota >= k, _sc_splat(nz[c], k), 0.0)
            scum.append(cs)
            sprev.append(cs - nz[c])
            carry = _sc_splat(cs, 15)

        acc = zeros16
        for t in range(2, _TMAX + 1):
            gidx = t - 2
            st_v = _sc_splat(scum[gidx // 16], gidx % 16)
            wsum = zeros16
            for c in range(4):
                hi = t - 2 - 16 * c
                if hi < 0:
                    continue
                term = g[c] * jnp.exp(beta_v * (sprev[c] - st_v))
                if hi >= 15:
                    wsum = wsum + term
                else:
                    wsum = wsum + jnp.where(iota <= hi, term, 0.0)
            acc = acc + jnp.where(jnp.float32(t) <= nf_v, wsum, 0.0)
        tot_v = _sc_splat(acc, 0)
        for k in range(1, 16):
            tot_v = tot_v + _sc_splat(acc, k)
        tot_v = tot_v * c_v

        o = jnp.where(iota == 0, tot_v,
                      jnp.where(iota == 1, nf_v,
                                jnp.where(iota == 2, c1_v, zeros16)))
        outv[pl.ds(0, 16)] = o
        pltpu.sync_copy(outv, out_hbm)


_sc_segment_kernel = functools.partial(
    pl.kernel,
    out_type=jax.ShapeDtypeStruct((16,), jnp.float32),
    mesh=plsc.VectorSubcoreMesh(core_axis_name="c", subcore_axis_name="s"),
    compiler_params=pltpu.CompilerParams(needs_layout_passes=False),
    scratch_types=[
        pltpu.VMEM((_EV_PER_SUB,), jnp.float32),
        pltpu.VMEM((_NBINS,), jnp.float32),
        pltpu.VMEM((16,), jnp.float32),
        pltpu.VMEM((_NSUB * _NBINS,), jnp.float32),
        pltpu.VMEM((16,), jnp.float32),
        pltpu.VMEM_SHARED((_NSUB * _NBINS,), jnp.float32),
    ],
)(_sc_segment_body)


def _etas_kernel(tr, tc, xr, xc, yr, yc, par, scv, o_loglik, o_lams1, o_lams2):
    lam0 = par[0, 0]
    c = par[0, 1]
    beta = par[0, 2]
    sx = par[0, 3]
    sy = par[0, 4]

    coef = c / (sx * sy * (2.0 * math.pi))
    inv_sx = 1.0 / (jnp.sqrt(jnp.float32(2.0)) * sx)
    inv_sy = 1.0 / (jnp.sqrt(jnp.float32(2.0)) * sy)
    nbeta = -beta

    tcv = tc[:, :]
    xcv = xc[:, :] * inv_sx
    ycv = yc[:, :] * inv_sy

    logsum = jnp.float32(0.0)
    for i in range(_NPAD // _CHUNK):
        ncol = (i + 1) * _CHUNK
        ta = tr[pl.ds(i * _CHUNK, _CHUNK), :]
        xa = xr[pl.ds(i * _CHUNK, _CHUNK), :] * inv_sx
        ya = yr[pl.ds(i * _CHUNK, _CHUNK), :] * inv_sy
        tb = tcv[:, :ncol]
        mask = tb < ta
        dt = jnp.where(mask, ta - tb, 1.0)
        r = 1.0 / dt
        dx = xa - xcv[:, :ncol]
        dy = ya - ycv[:, :ncol]
        s = dx * dx + dy * dy
        expo = nbeta * dt - s * r
        w = jnp.where(mask, jnp.exp(expo) * r, 0.0)
        lam = coef * jnp.sum(w, axis=1, keepdims=True)
        lmask = (ta >= 2.0) & (ta <= float(_TMAX))
        lam_safe = jnp.where(lmask, lam, 1.0)
        logsum = logsum + jnp.sum(jnp.where(lmask, jnp.log(lam_safe), 0.0))

    sc = scv[:, :]
    lane = lax.broadcasted_iota(jnp.int32, (1, 16), 1)
    total = jnp.sum(jnp.where(lane == 0, sc, 0.0))
    n_f = jnp.sum(jnp.where(lane == 1, sc, 0.0))
    count1 = jnp.sum(jnp.where(lane == 2, sc, 0.0))

    lams1 = count1 * jnp.log(lam0) + logsum
    lams2 = lam0 * n_f + total
    o_loglik[:, :] = jnp.reshape(lams1 - lams2, (1, 1))
    o_lams1[:, :] = jnp.reshape(lams1, (1, 1))
    o_lams2[:, :] = jnp.reshape(lams2, (1, 1))


def kernel(obs, Lambda0, C, Beta, Sigmax, Sigmay):
    times = obs[:, 0]
    x = obs[:, 1]
    y = obs[:, 2]
    pad = _NPAD - _N
    tpad = jnp.pad(times, (0, pad), constant_values=_PADVAL)
    xpad = jnp.pad(x, (0, pad), constant_values=0.0)
    ypad = jnp.pad(y, (0, pad), constant_values=0.0)

    tr = tpad[:, None]
    tc = tpad[None, :]
    xr = xpad[:, None]
    xc = xpad[None, :]
    yr = ypad[:, None]
    yc = ypad[None, :]
    par = jnp.stack([Lambda0, C, Beta, Sigmax, Sigmay,
                     jnp.float32(0.0), jnp.float32(0.0), jnp.float32(0.0)])[None, :]
    par16 = jnp.concatenate(
        [par[0], jnp.zeros((8,), jnp.float32)], axis=0)

    sc_out = _sc_segment_kernel(tpad, par16)

    out_shape = [jax.ShapeDtypeStruct((1, 1), jnp.float32)] * 3
    loglik, lams1, lams2 = pl.pallas_call(
        _etas_kernel,
        out_shape=out_shape,
    )(tr, tc, xr, xc, yr, yc, par, sc_out[None, :])
    return (loglik[0, 0], lams1[0, 0], lams2[0, 0])
